# Initial kernel scaffold; baseline (speedup 1.0000x reference)
import jax
import jax.numpy as jnp
from jax.experimental import pallas as pl


def kernel(hidden_states, router_weight, gate_up_proj, down_proj):
    raise NotImplementedError("write your pallas kernel here")


# speedup vs baseline: 4.3637x; 4.3637x over previous
import functools

import jax
import jax.numpy as jnp
from jax.experimental import pallas as pl

_HIDDEN = 32
_INTER = 16
_EXPERTS = 4
_BLOCK = 2048


def _moe_body(x_ref, wr_ref, wg_ref, wu_ref, wd_ref, o_ref):
    x = x_ref[...]

    logits = jnp.dot(x, wr_ref[...], preferred_element_type=jnp.float32)
    mx = jnp.max(logits, axis=-1, keepdims=True)
    u = jnp.exp(logits - mx)

    eidx = jax.lax.broadcasted_iota(jnp.int32, u.shape, 1)
    m1 = jnp.max(u, axis=-1, keepdims=True)
    i1 = jnp.min(jnp.where(u == m1, eidx, _EXPERTS), axis=-1, keepdims=True)
    u2 = jnp.where(eidx == i1, -jnp.inf, u)
    m2 = jnp.max(u2, axis=-1, keepdims=True)
    i2 = jnp.min(jnp.where(u2 == m2, eidx, _EXPERTS), axis=-1, keepdims=True)
    inv_denom = 1.0 / (m1 + m2)

    cexp = jax.lax.broadcasted_iota(jnp.int32, (x.shape[0], _EXPERTS * _INTER), 1)
    cexp = jax.lax.shift_right_logical(cexp, 4)
    rw = (jnp.where(cexp == i1, m1, 0.0) + jnp.where(cexp == i2, m2, 0.0)) * inv_denom

    gate = jnp.dot(x, wg_ref[...], preferred_element_type=jnp.float32)
    up = jnp.dot(x, wu_ref[...], preferred_element_type=jnp.float32)
    h = gate * (1.0 / (1.0 + jnp.exp(-gate))) * up * rw
    o_ref[...] = jnp.dot(h, wd_ref[...], preferred_element_type=jnp.float32)


@jax.jit
def kernel(hidden_states, router_weight, gate_up_proj, down_proj):
    batch, seq, hidden = hidden_states.shape
    n_tokens = batch * seq
    x = hidden_states.reshape(n_tokens, hidden)

    wr = router_weight.T.astype(jnp.float32)
    gate_w = gate_up_proj[:, :_INTER, :]
    up_w = gate_up_proj[:, _INTER:, :]
    wg = jnp.transpose(gate_w, (2, 0, 1)).reshape(hidden, _EXPERTS * _INTER)
    wu = jnp.transpose(up_w, (2, 0, 1)).reshape(hidden, _EXPERTS * _INTER)
    wd = jnp.transpose(down_proj, (0, 2, 1)).reshape(_EXPERTS * _INTER, hidden)

    grid = (n_tokens // _BLOCK,)
    out = pl.pallas_call(
        _moe_body,
        grid=grid,
        in_specs=[
            pl.BlockSpec((_BLOCK, hidden), lambda i: (i, 0)),
            pl.BlockSpec(wr.shape, lambda i: (0, 0)),
            pl.BlockSpec(wg.shape, lambda i: (0, 0)),
            pl.BlockSpec(wu.shape, lambda i: (0, 0)),
            pl.BlockSpec(wd.shape, lambda i: (0, 0)),
        ],
        out_specs=pl.BlockSpec((_BLOCK, hidden), lambda i: (i, 0)),
        out_shape=jax.ShapeDtypeStruct((n_tokens, hidden), jnp.float32),
    )(x, wr, wg, wu, wd)
    return out.reshape(batch, seq, hidden)


# speedup vs baseline: 6.2078x; 1.4226x over previous
import functools

import jax
import jax.numpy as jnp
from jax.experimental import pallas as pl

_HIDDEN = 32
_INTER = 16
_EXPERTS = 4
_BLOCK = 2048

_RHS_T = (((1,), (1,)), ((), ()))


def _moe_body(x_ref, wr_ref, wg_ref, wu_ref, wd_ref, o_ref):
    x = x_ref[...]
    f32 = jnp.float32

    l4 = jax.lax.dot_general(wr_ref[...], x, _RHS_T, preferred_element_type=f32)

    l1 = jnp.max(l4, axis=0, keepdims=True)
    eidx4 = jax.lax.broadcasted_iota(jnp.int32, l4.shape, 0)
    i1 = jnp.min(jnp.where(l4 == l1, eidx4, _EXPERTS), axis=0, keepdims=True)
    lm = jnp.where(eidx4 == i1, -jnp.inf, l4)
    l2 = jnp.max(lm, axis=0, keepdims=True)
    i2 = jnp.min(jnp.where(lm == l2, eidx4, _EXPERTS), axis=0, keepdims=True)

    e21 = jnp.exp(l2 - l1)
    s = e21 / (1.0 + e21)

    g = jax.lax.dot_general(wg_ref[...], x, _RHS_T, preferred_element_type=f32)
    u = jax.lax.dot_general(wu_ref[...], x, _RHS_T, preferred_element_type=f32)

    eidx64 = jax.lax.broadcasted_iota(jnp.int32, g.shape, 0)
    eidx64 = jax.lax.shift_right_logical(eidx64, 4)
    rw = jnp.where(eidx64 == i1, 1.0 - s, jnp.where(eidx64 == i2, s, 0.0))

    h = g * (1.0 / (1.0 + jnp.exp(-g))) * u * rw
    out_t = jax.lax.dot_general(wd_ref[...], h, (((1,), (0,)), ((), ())),
                                preferred_element_type=f32)
    o_ref[...] = out_t.T


@jax.jit
def kernel(hidden_states, router_weight, gate_up_proj, down_proj):
    batch, seq, hidden = hidden_states.shape
    n_tokens = batch * seq
    x = hidden_states.reshape(n_tokens, hidden)

    wr = router_weight.astype(jnp.float32)
    gate_w = gate_up_proj[:, :_INTER, :]
    up_w = gate_up_proj[:, _INTER:, :]
    wg = gate_w.reshape(_EXPERTS * _INTER, hidden)
    wu = up_w.reshape(_EXPERTS * _INTER, hidden)
    wd = jnp.transpose(down_proj, (1, 0, 2)).reshape(hidden, _EXPERTS * _INTER)

    grid = (n_tokens // _BLOCK,)
    out = pl.pallas_call(
        _moe_body,
        grid=grid,
        in_specs=[
            pl.BlockSpec((_BLOCK, hidden), lambda i: (i, 0)),
            pl.BlockSpec(wr.shape, lambda i: (0, 0)),
            pl.BlockSpec(wg.shape, lambda i: (0, 0)),
            pl.BlockSpec(wu.shape, lambda i: (0, 0)),
            pl.BlockSpec(wd.shape, lambda i: (0, 0)),
        ],
        out_specs=pl.BlockSpec((_BLOCK, hidden), lambda i: (i, 0)),
        out_shape=jax.ShapeDtypeStruct((n_tokens, hidden), jnp.float32),
    )(x, wr, wg, wu, wd)
    return out.reshape(batch, seq, hidden)


# speedup vs baseline: 7.0084x; 1.1290x over previous
import functools

import jax
import jax.numpy as jnp
from jax.experimental import pallas as pl

_HIDDEN = 32
_INTER = 16
_EXPERTS = 4
_BLOCK = 8192

_RHS_T = (((1,), (1,)), ((), ()))


def _moe_body(x_ref, wr_ref, wg_ref, wu_ref, wd_ref, o_ref):
    x = x_ref[...]
    f32 = jnp.float32

    l4 = jax.lax.dot_general(wr_ref[...], x, _RHS_T, preferred_element_type=f32)

    l1 = jnp.max(l4, axis=0, keepdims=True)
    eidx4 = jax.lax.broadcasted_iota(jnp.int32, l4.shape, 0)
    i1 = jnp.min(jnp.where(l4 == l1, eidx4, _EXPERTS), axis=0, keepdims=True)
    lm = jnp.where(eidx4 == i1, -jnp.inf, l4)
    l2 = jnp.max(lm, axis=0, keepdims=True)
    i2 = jnp.min(jnp.where(lm == l2, eidx4, _EXPERTS), axis=0, keepdims=True)

    e21 = jnp.exp(l2 - l1)
    s = e21 / (1.0 + e21)

    g = jax.lax.dot_general(wg_ref[...], x, _RHS_T, preferred_element_type=f32)
    u = jax.lax.dot_general(wu_ref[...], x, _RHS_T, preferred_element_type=f32)

    eidx64 = jax.lax.broadcasted_iota(jnp.int32, g.shape, 0)
    eidx64 = jax.lax.shift_right_logical(eidx64, 4)
    rw = jnp.where(eidx64 == i1, 1.0 - s, jnp.where(eidx64 == i2, s, 0.0))

    h = g * (1.0 / (1.0 + jnp.exp(-g))) * u * rw
    out_t = jax.lax.dot_general(wd_ref[...], h, (((1,), (0,)), ((), ())),
                                preferred_element_type=f32)
    o_ref[...] = out_t.T


@jax.jit
def kernel(hidden_states, router_weight, gate_up_proj, down_proj):
    batch, seq, hidden = hidden_states.shape
    n_tokens = batch * seq
    x = hidden_states.reshape(n_tokens, hidden)

    wr = router_weight.astype(jnp.float32)
    gate_w = gate_up_proj[:, :_INTER, :]
    up_w = gate_up_proj[:, _INTER:, :]
    wg = gate_w.reshape(_EXPERTS * _INTER, hidden)
    wu = up_w.reshape(_EXPERTS * _INTER, hidden)
    wd = jnp.transpose(down_proj, (1, 0, 2)).reshape(hidden, _EXPERTS * _INTER)

    grid = (n_tokens // _BLOCK,)
    out = pl.pallas_call(
        _moe_body,
        grid=grid,
        in_specs=[
            pl.BlockSpec((_BLOCK, hidden), lambda i: (i, 0)),
            pl.BlockSpec(wr.shape, lambda i: (0, 0)),
            pl.BlockSpec(wg.shape, lambda i: (0, 0)),
            pl.BlockSpec(wu.shape, lambda i: (0, 0)),
            pl.BlockSpec(wd.shape, lambda i: (0, 0)),
        ],
        out_specs=pl.BlockSpec((_BLOCK, hidden), lambda i: (i, 0)),
        out_shape=jax.ShapeDtypeStruct((n_tokens, hidden), jnp.float32),
    )(x, wr, wg, wu, wd)
    return out.reshape(batch, seq, hidden)
